# baseline (speedup 1.0000x reference)
import functools

import jax
import jax.numpy as jnp
from jax import lax
from jax.experimental import pallas as pl
from jax.experimental.pallas import tpu as pltpu
from jax.experimental.pallas import tpu_sc as plsc

IY = 225
IX = 224
B, H, W, C = 32, 450, 449, 24
NCH = 6
NOUT = B * NCH


def _sc_gather(x, chan16):
    mesh = plsc.VectorSubcoreMesh(core_axis_name="c", subcore_axis_name="s")

    @functools.partial(
        pl.kernel,
        mesh=mesh,
        out_type=jax.ShapeDtypeStruct((NOUT,), jnp.float32),
        scratch_types=[
            pltpu.VMEM((B, C), jnp.float32),
            pltpu.VMEM((16,), jnp.int32),
            pltpu.VMEM((NOUT,), jnp.float32),
        ],
        compiler_params=pltpu.CompilerParams(needs_layout_passes=False),
    )
    def k(x_hbm, chan_hbm, out_hbm, point_v, chan_v, out_v):
        wid = lax.axis_index("s") * 2 + lax.axis_index("c")

        @pl.when(wid == 0)
        def _():
            pltpu.sync_copy(x_hbm.at[:, IY, IX], point_v)
            pltpu.sync_copy(chan_hbm, chan_v)
            lanes = lax.iota(jnp.int32, 16)
            six = jnp.full((16,), NCH, jnp.int32)
            for g in range(NOUT // 16):
                f = lanes + g * 16
                b = lax.div(f, six)
                c = plsc.load_gather(chan_v, [lax.rem(f, six)])
                out_v[pl.ds(g * 16, 16)] = plsc.load_gather(point_v, [b, c])
            pltpu.sync_copy(out_v, out_hbm)

    return k(x, chan16)


def kernel(x, chan):
    chan16 = jnp.zeros((16,), jnp.int32).at[:NCH].set(chan.astype(jnp.int32))
    return _sc_gather(x, chan16).reshape(B, NCH)
